# baseline (speedup 1.0000x reference)
import jax
import jax.numpy as jnp
from jax.experimental import pallas as pl
from jax.experimental.pallas import tpu as pltpu

B, N, M, D = 16, 2048, 2048, 3
NT = 8
TN = N // NT


def _chamfer_body(pred_ref, tgtt_ref, rowsum_ref, colsum_ref, colmin_ref):
    b = pl.program_id(0)
    i = pl.program_id(1)

    @pl.when((b == 0) & (i == 0))
    def _init():
        rowsum_ref[0, 0] = 0.0
        colsum_ref[0, 0] = 0.0

    x = pred_ref[0]
    y = tgtt_ref[0]
    xr = x.astype(jnp.bfloat16).astype(jnp.float32)
    yr = y.astype(jnp.bfloat16).astype(jnp.float32)
    xy = (xr[:, 0:1] * yr[0:1, :]
          + xr[:, 1:2] * yr[1:2, :]
          + xr[:, 2:3] * yr[2:3, :])
    xx = jnp.sum(x * x, axis=1, keepdims=True)
    yy = jnp.sum(y * y, axis=0, keepdims=True)
    d = jnp.maximum(xx + yy - 2.0 * xy, 0.0)

    rowsum_ref[0, 0] += jnp.sum(jnp.min(d, axis=1))

    colpart = jnp.min(d, axis=0, keepdims=True)

    @pl.when(i == 0)
    def _first():
        colmin_ref[...] = colpart

    @pl.when(i > 0)
    def _rest():
        colmin_ref[...] = jnp.minimum(colmin_ref[...], colpart)

    @pl.when(i == NT - 1)
    def _finish():
        colsum_ref[0, 0] += jnp.sum(colmin_ref[...])


def kernel(predictions, targets):
    tgt_t = jnp.transpose(targets, (0, 2, 1))
    rowsum, colsum = pl.pallas_call(
        _chamfer_body,
        grid=(B, NT),
        in_specs=[
            pl.BlockSpec((1, TN, D), lambda b, i: (b, i, 0)),
            pl.BlockSpec((1, D, M), lambda b, i: (b, 0, 0)),
        ],
        out_specs=[
            pl.BlockSpec((1, 1), lambda b, i: (0, 0), memory_space=pltpu.SMEM),
            pl.BlockSpec((1, 1), lambda b, i: (0, 0), memory_space=pltpu.SMEM),
        ],
        out_shape=[
            jax.ShapeDtypeStruct((1, 1), jnp.float32),
            jax.ShapeDtypeStruct((1, 1), jnp.float32),
        ],
        scratch_shapes=[pltpu.VMEM((1, M), jnp.float32)],
        compiler_params=pltpu.CompilerParams(
            dimension_semantics=("arbitrary", "arbitrary"),
        ),
    )(predictions, tgt_t)
    return rowsum[0, 0] / (B * N) + colsum[0, 0] / (B * M)


# speedup vs baseline: 1.2535x; 1.2535x over previous
import jax
import jax.numpy as jnp
from jax.experimental import pallas as pl
from jax.experimental.pallas import tpu as pltpu

B, N, M, D = 16, 2048, 2048, 3
KP = 16
NT = 8
TN = N // NT


def _chamfer_body(pred_ref, tgtt_ref, xm2_ref, ytb_ref,
                  rowsum_ref, colsum_ref, colmin_ref):
    b = pl.program_id(0)
    i = pl.program_id(1)

    @pl.when((b == 0) & (i == 0))
    def _init():
        rowsum_ref[0, 0] = 0.0
        colsum_ref[0, 0] = 0.0

    x = pred_ref[0]
    yt = tgtt_ref[0]
    xx = jnp.sum(x * x, axis=1, keepdims=True)
    yy = jnp.sum(yt * yt, axis=0, keepdims=True)

    t = jax.lax.dot_general(
        xm2_ref[0], ytb_ref[0],
        dimension_numbers=(((1,), (0,)), ((), ())),
        preferred_element_type=jnp.float32,
    )

    rmin = jnp.min(t + yy, axis=1, keepdims=True)
    rowsum_ref[0, 0] += jnp.sum(jnp.maximum(rmin + xx, 0.0))

    colpart = jnp.min(t + xx, axis=0, keepdims=True)

    @pl.when(i == 0)
    def _first():
        colmin_ref[...] = colpart

    @pl.when(i > 0)
    def _rest():
        colmin_ref[...] = jnp.minimum(colmin_ref[...], colpart)

    @pl.when(i == NT - 1)
    def _finish():
        colsum_ref[0, 0] += jnp.sum(jnp.maximum(colmin_ref[...] + yy, 0.0))


def kernel(predictions, targets):
    tgt_t = jnp.transpose(targets, (0, 2, 1))
    xm2 = jnp.pad((predictions * -2.0).astype(jnp.bfloat16),
                  ((0, 0), (0, 0), (0, KP - D)))
    ytb = jnp.pad(tgt_t.astype(jnp.bfloat16),
                  ((0, 0), (0, KP - D), (0, 0)))
    rowsum, colsum = pl.pallas_call(
        _chamfer_body,
        grid=(B, NT),
        in_specs=[
            pl.BlockSpec((1, TN, D), lambda b, i: (b, i, 0)),
            pl.BlockSpec((1, D, M), lambda b, i: (b, 0, 0)),
            pl.BlockSpec((1, TN, KP), lambda b, i: (b, i, 0)),
            pl.BlockSpec((1, KP, M), lambda b, i: (b, 0, 0)),
        ],
        out_specs=[
            pl.BlockSpec((1, 1), lambda b, i: (0, 0), memory_space=pltpu.SMEM),
            pl.BlockSpec((1, 1), lambda b, i: (0, 0), memory_space=pltpu.SMEM),
        ],
        out_shape=[
            jax.ShapeDtypeStruct((1, 1), jnp.float32),
            jax.ShapeDtypeStruct((1, 1), jnp.float32),
        ],
        scratch_shapes=[pltpu.VMEM((1, M), jnp.float32)],
        compiler_params=pltpu.CompilerParams(
            dimension_semantics=("arbitrary", "arbitrary"),
        ),
    )(predictions, tgt_t, xm2, ytb)
    return rowsum[0, 0] / (B * N) + colsum[0, 0] / (B * M)


# speedup vs baseline: 2.1684x; 1.7299x over previous
import jax
import jax.numpy as jnp
from jax.experimental import pallas as pl
from jax.experimental.pallas import tpu as pltpu

B, N, M, D = 16, 2048, 2048, 3
KP = 16


def _chamfer_body(pred_ref, tgtt_ref, xm2_ref, ytb_ref,
                  rowsum_ref, colsum_ref):
    b = pl.program_id(0)

    @pl.when(b == 0)
    def _init():
        rowsum_ref[0, 0] = 0.0
        colsum_ref[0, 0] = 0.0

    x = pred_ref[0]
    yt = tgtt_ref[0]
    xx = jnp.sum(x * x, axis=1, keepdims=True)
    yy = jnp.sum(yt * yt, axis=0, keepdims=True)

    t = jax.lax.dot_general(
        xm2_ref[0], ytb_ref[0],
        dimension_numbers=(((1,), (0,)), ((), ())),
        preferred_element_type=jnp.float32,
    )

    rmin = jnp.min(t + yy, axis=1, keepdims=True)
    rowsum_ref[0, 0] += jnp.sum(jnp.maximum(rmin + xx, 0.0))

    cmin = jnp.min(t + xx, axis=0, keepdims=True)
    colsum_ref[0, 0] += jnp.sum(jnp.maximum(cmin + yy, 0.0))


def kernel(predictions, targets):
    tgt_t = jnp.transpose(targets, (0, 2, 1))
    xm2 = jnp.pad((predictions * -2.0).astype(jnp.bfloat16),
                  ((0, 0), (0, 0), (0, KP - D)))
    ytb = jnp.pad(tgt_t.astype(jnp.bfloat16),
                  ((0, 0), (0, KP - D), (0, 0)))
    rowsum, colsum = pl.pallas_call(
        _chamfer_body,
        grid=(B,),
        in_specs=[
            pl.BlockSpec((1, N, D), lambda b: (b, 0, 0)),
            pl.BlockSpec((1, D, M), lambda b: (b, 0, 0)),
            pl.BlockSpec((1, N, KP), lambda b: (b, 0, 0)),
            pl.BlockSpec((1, KP, M), lambda b: (b, 0, 0)),
        ],
        out_specs=[
            pl.BlockSpec((1, 1), lambda b: (0, 0), memory_space=pltpu.SMEM),
            pl.BlockSpec((1, 1), lambda b: (0, 0), memory_space=pltpu.SMEM),
        ],
        out_shape=[
            jax.ShapeDtypeStruct((1, 1), jnp.float32),
            jax.ShapeDtypeStruct((1, 1), jnp.float32),
        ],
        compiler_params=pltpu.CompilerParams(
            dimension_semantics=("arbitrary",),
        ),
    )(predictions, tgt_t, xm2, ytb)
    return rowsum[0, 0] / (B * N) + colsum[0, 0] / (B * M)


# speedup vs baseline: 2.9077x; 1.3410x over previous
import jax
import jax.numpy as jnp
from jax.experimental import pallas as pl
from jax.experimental.pallas import tpu as pltpu

B, N, M, D = 16, 2048, 2048, 3
KP = 16


def _chamfer_body(xm_ref, ymt_ref, rowsum_ref, colsum_ref):
    b = pl.program_id(0)

    @pl.when(b == 0)
    def _init():
        rowsum_ref[0, 0] = 0.0
        colsum_ref[0, 0] = 0.0

    d = jax.lax.dot_general(
        xm_ref[0], ymt_ref[0],
        dimension_numbers=(((1,), (0,)), ((), ())),
        preferred_element_type=jnp.float32,
    )

    rmin = jnp.min(d, axis=1)
    rowsum_ref[0, 0] += jnp.sum(jnp.maximum(rmin, 0.0))
    cmin = jnp.min(d, axis=0)
    colsum_ref[0, 0] += jnp.sum(jnp.maximum(cmin, 0.0))


def _split3_bf16(v):
    hi = v.astype(jnp.bfloat16)
    r1 = v - hi.astype(jnp.float32)
    mid = r1.astype(jnp.bfloat16)
    lo = (r1 - mid.astype(jnp.float32)).astype(jnp.bfloat16)
    return hi, mid, lo


def kernel(predictions, targets):
    xx = jnp.sum(predictions * predictions, axis=2, keepdims=True)
    yy = jnp.sum(targets * targets, axis=2, keepdims=True)
    xxh, xxm, xxl = _split3_bf16(xx)
    yyh, yym, yyl = _split3_bf16(yy)
    ones_x = jnp.ones((B, N, 1), jnp.bfloat16)
    ones_y = jnp.ones((B, M, 1), jnp.bfloat16)
    zeros_x = jnp.zeros((B, N, KP - 9), jnp.bfloat16)
    zeros_y = jnp.zeros((B, M, KP - 9), jnp.bfloat16)
    xm = jnp.concatenate(
        [(predictions * -2.0).astype(jnp.bfloat16),
         xxh, xxm, xxl, ones_x, ones_x, ones_x, zeros_x], axis=2)
    ym = jnp.concatenate(
        [targets.astype(jnp.bfloat16),
         ones_y, ones_y, ones_y, yyh, yym, yyl, zeros_y], axis=2)
    ymt = jnp.transpose(ym, (0, 2, 1))

    rowsum, colsum = pl.pallas_call(
        _chamfer_body,
        grid=(B,),
        in_specs=[
            pl.BlockSpec((1, N, KP), lambda b: (b, 0, 0)),
            pl.BlockSpec((1, KP, M), lambda b: (b, 0, 0)),
        ],
        out_specs=[
            pl.BlockSpec((1, 1), lambda b: (0, 0), memory_space=pltpu.SMEM),
            pl.BlockSpec((1, 1), lambda b: (0, 0), memory_space=pltpu.SMEM),
        ],
        out_shape=[
            jax.ShapeDtypeStruct((1, 1), jnp.float32),
            jax.ShapeDtypeStruct((1, 1), jnp.float32),
        ],
        compiler_params=pltpu.CompilerParams(
            dimension_semantics=("arbitrary",),
        ),
    )(xm, ymt)
    return rowsum[0, 0] / (B * N) + colsum[0, 0] / (B * M)
